# baseline (speedup 1.0000x reference)
import jax
import jax.numpy as jnp
from jax.experimental import pallas as pl
from jax.experimental.pallas import tpu as pltpu

NUM_EXPERTS = 8
TOP_K = 2
BLK = 128
ROWS = 32
LANES = 128


def _router(e_ref, dst_ref, bexp_ref):
    e2d = e_ref[...]
    fBLK = float(BLK)

    r32 = jax.lax.broadcasted_iota(jnp.int32, (ROWS, ROWS), 0)
    c32 = jax.lax.broadcasted_iota(jnp.int32, (ROWS, ROWS), 1)
    L32 = (r32 >= c32).astype(jnp.float32)
    rl = jax.lax.broadcasted_iota(jnp.int32, (LANES, LANES), 0)
    cl = jax.lax.broadcasted_iota(jnp.int32, (LANES, LANES), 1)
    Us = (rl < cl).astype(jnp.float32)

    csum_v = []
    ohs = []
    col_rows = []
    for e in range(NUM_EXPERTS):
        oh = (e2d == e).astype(jnp.float32)
        cv = jnp.dot(L32, oh, preferred_element_type=jnp.float32)
        ohs.append(oh)
        csum_v.append(cv)
        col_rows.append(cv[ROWS - 1:ROWS, :])
    C = jnp.concatenate(col_rows, axis=0)
    P8 = jnp.dot(C, Us, preferred_element_type=jnp.float32)
    counts = jnp.sum(C, axis=1, keepdims=True)
    ccounts = jnp.floor((counts + (fBLK - 1.0)) * (1.0 / fBLK)) * fBLK
    r8 = jax.lax.broadcasted_iota(jnp.int32, (NUM_EXPERTS, NUM_EXPERTS), 0)
    c8 = jax.lax.broadcasted_iota(jnp.int32, (NUM_EXPERTS, NUM_EXPERTS), 1)
    L8s = (r8 > c8).astype(jnp.float32)
    pstart = jnp.dot(L8s, ccounts, preferred_element_type=jnp.float32)

    dstf = jnp.zeros((ROWS, LANES), jnp.float32)
    iota_l = jax.lax.broadcasted_iota(jnp.int32, (1, LANES), 1).astype(jnp.float32)
    bexp = jnp.zeros((1, LANES), jnp.float32)
    covered = jnp.zeros((1, LANES), jnp.float32)
    for e in range(NUM_EXPERTS):
        ps = pstart[e, 0]
        dstf = dstf + ohs[e] * (ps + P8[e:e + 1, :] + csum_v[e] - 1.0)
        bs = ps * (1.0 / fBLK)
        nb = ccounts[e, 0] * (1.0 / fBLK)
        mask = jnp.where((iota_l >= bs) & (iota_l < bs + nb), 1.0, 0.0)
        bexp = bexp + float(e) * mask
        covered = covered + mask
    bexp = bexp + float(NUM_EXPERTS - 1) * (1.0 - covered)

    dst_ref[...] = dstf.astype(jnp.int32)
    out = jnp.concatenate(
        [bexp, jnp.zeros((NUM_EXPERTS - 1, LANES), jnp.float32)], axis=0)
    bexp_ref[...] = out.astype(jnp.int32)


def _dispatch(dst_ref, tok_ref):
    def clear(s, carry):
        tok_ref[s] = ROWS * LANES
        return carry

    jax.lax.fori_loop(0, tok_ref.shape[0], clear, 0, unroll=16)

    def place(p, carry):
        d = dst_ref[p // LANES, p % LANES]
        tok_ref[d] = p
        return carry

    jax.lax.fori_loop(0, ROWS * LANES, place, 0, unroll=16)


def _routing_metadata(expert_indices, expert_weights, T):
    P = T * TOP_K
    PP = P + NUM_EXPERTS * BLK
    NB = PP // BLK
    e2d = expert_indices.reshape(ROWS, LANES).astype(jnp.int32)
    w2d = expert_weights.reshape(ROWS, LANES)

    dst2d, bexp8 = pl.pallas_call(
        _router,
        out_shape=(
            jax.ShapeDtypeStruct((ROWS, LANES), jnp.int32),
            jax.ShapeDtypeStruct((NUM_EXPERTS, LANES), jnp.int32),
        ),
    )(e2d)

    tok = pl.pallas_call(
        _dispatch,
        grid_spec=pltpu.PrefetchScalarGridSpec(
            num_scalar_prefetch=1,
            grid=(1,),
            in_specs=[],
            out_specs=pl.BlockSpec(memory_space=pltpu.SMEM),
        ),
        out_shape=jax.ShapeDtypeStruct((PP,), jnp.int32),
    )(dst2d)
    w_pad = jnp.concatenate(
        [w2d, jnp.zeros((NB - ROWS, LANES), jnp.float32)], axis=0)
    return tok, w_pad, bexp8, NB


def _moe_block(be_ref, tok_ref, w_ref, x_ref, gup_ref, dp_ref, o_ref, xs, ys):
    b = pl.program_id(0)

    @pl.when(b == 0)
    def _init():
        o_ref[...] = jnp.zeros(o_ref.shape, o_ref.dtype)

    base = b * BLK
    T = x_ref.shape[0]

    def gather_one(i, carry):
        p = tok_ref[base + i]
        t = jnp.minimum(p // TOP_K, T - 1)
        xs[i, :] = x_ref[t, :]
        return carry

    jax.lax.fori_loop(0, BLK, gather_one, 0, unroll=8)

    x = xs[...].astype(jnp.bfloat16)
    gup = gup_ref[0].astype(jnp.bfloat16)
    half = gup.shape[1] // 2
    gu = jnp.dot(x, gup, preferred_element_type=jnp.float32)
    g = gu[:, :half]
    u = gu[:, half:]
    act = (g * jax.nn.sigmoid(g) * u).astype(jnp.bfloat16)
    y = jnp.dot(act, dp_ref[0].astype(jnp.bfloat16),
                preferred_element_type=jnp.float32)
    ys[...] = y

    def scatter_one(i, carry):
        p = tok_ref[base + i]
        t = jnp.minimum(p // TOP_K, T - 1)
        w = w_ref[p // LANES, p % LANES]
        o_ref[t, :] += ys[i, :] * w
        return carry

    jax.lax.fori_loop(0, BLK, scatter_one, 0, unroll=8)


def kernel(hidden_states, gate_up_proj, down_proj, expert_indices, expert_weights):
    B, S, H = hidden_states.shape
    T = B * S
    E, _, I2 = gate_up_proj.shape
    I = I2 // 2
    flat = hidden_states.reshape(T, H)

    tok, w_pad, bexp8, NB = _routing_metadata(expert_indices, expert_weights, T)

    grid_spec = pltpu.PrefetchScalarGridSpec(
        num_scalar_prefetch=3,
        grid=(NB,),
        in_specs=[
            pl.BlockSpec((T, H), lambda b, be, tk, w: (0, 0)),
            pl.BlockSpec((1, H, I2), lambda b, be, tk, w: (be[0, b], 0, 0)),
            pl.BlockSpec((1, I, H), lambda b, be, tk, w: (be[0, b], 0, 0)),
        ],
        out_specs=pl.BlockSpec((T, H), lambda b, be, tk, w: (0, 0)),
        scratch_shapes=[
            pltpu.VMEM((BLK, H), jnp.float32),
            pltpu.VMEM((BLK, H), jnp.float32),
        ],
    )
    out = pl.pallas_call(
        _moe_block,
        grid_spec=grid_spec,
        out_shape=jax.ShapeDtypeStruct((T, H), jnp.float32),
    )(
        bexp8,
        tok,
        w_pad,
        flat,
        gate_up_proj,
        down_proj,
    )
    return out.reshape(B, S, H)


# speedup vs baseline: 1.6509x; 1.6509x over previous
import jax
import jax.numpy as jnp
from jax.experimental import pallas as pl
from jax.experimental.pallas import tpu as pltpu

NUM_EXPERTS = 8
TOP_K = 2
NCORES = 2
BT = 512


def _moe_expert(idx_ref, ew_ref, x_ref, gup_ref, dp_ref, o_ref):
    e = pl.program_id(1)
    gup = gup_ref[0].astype(jnp.bfloat16)
    dp = dp_ref[0].astype(jnp.bfloat16)
    TC = x_ref.shape[0]
    half = gup.shape[1] // 2
    for t in range(TC // BT):
        sl = pl.ds(t * BT, BT)
        x = x_ref[sl, :].astype(jnp.bfloat16)
        gu = jnp.dot(x, gup, preferred_element_type=jnp.float32)
        g = gu[:, :half]
        u = gu[:, half:]
        act = g * jax.nn.sigmoid(g) * u
        eo = jnp.dot(act.astype(jnp.bfloat16), dp,
                     preferred_element_type=jnp.float32)
        w = jnp.sum(jnp.where(idx_ref[sl, :] == e, ew_ref[sl, :], 0.0), axis=1)
        contrib = eo * w[:, None]

        @pl.when(e == 0)
        def _init():
            o_ref[sl, :] = contrib

        @pl.when(e > 0)
        def _acc():
            o_ref[sl, :] += contrib


def kernel(hidden_states, gate_up_proj, down_proj, expert_indices, expert_weights):
    B, S, H = hidden_states.shape
    T = B * S
    E, _, I2 = gate_up_proj.shape
    I = I2 // 2
    TC = T // NCORES
    flat = hidden_states.reshape(T, H)

    out = pl.pallas_call(
        _moe_expert,
        grid=(NCORES, E),
        in_specs=[
            pl.BlockSpec((TC, TOP_K), lambda c, e: (c, 0)),
            pl.BlockSpec((TC, TOP_K), lambda c, e: (c, 0)),
            pl.BlockSpec((TC, H), lambda c, e: (c, 0)),
            pl.BlockSpec((1, H, I2), lambda c, e: (e, 0, 0)),
            pl.BlockSpec((1, I, H), lambda c, e: (e, 0, 0)),
        ],
        out_specs=pl.BlockSpec((TC, H), lambda c, e: (c, 0)),
        out_shape=jax.ShapeDtypeStruct((T, H), jnp.float32),
        compiler_params=pltpu.CompilerParams(
            dimension_semantics=("parallel", "arbitrary")),
    )(
        expert_indices,
        expert_weights,
        flat,
        gate_up_proj,
        down_proj,
    )
    return out.reshape(B, S, H)
